# Initial kernel scaffold; baseline (speedup 1.0000x reference)
import jax
import jax.numpy as jnp
from jax.experimental import pallas as pl


def kernel(token_ids, table):
    raise NotImplementedError("write your pallas kernel here")


# speedup vs baseline: 1.4442x; 1.4442x over previous
import functools

import jax
import jax.numpy as jnp
from jax import lax
from jax.experimental import pallas as pl
from jax.experimental.pallas import tpu as pltpu
from jax.experimental.pallas import tpu_sc as plsc

VOCAB = 256
DIM = 768
CHUNK = 128


def _make_kernel(num_tokens: int):
  info = plsc.get_sparse_core_info()
  nc, ns = info.num_cores, info.num_subcores
  nw = nc * ns
  assert num_tokens % (nw * CHUNK) == 0
  per_w = num_tokens // nw
  n_chunks = per_w // CHUNK

  mesh = plsc.VectorSubcoreMesh(core_axis_name="c", subcore_axis_name="s")

  @functools.partial(
      pl.kernel,
      out_type=jax.ShapeDtypeStruct((num_tokens, DIM), jnp.float32),
      mesh=mesh,
      scratch_types=[
          pltpu.VMEM((CHUNK,), jnp.int32),
          pltpu.VMEM((CHUNK, DIM), jnp.float32),
          pltpu.SemaphoreType.DMA,
      ],
  )
  def gather_kernel(ids_hbm, table_hbm, out_hbm, idx_v, rows_v, sem):
    c = lax.axis_index("c")
    s = lax.axis_index("s")
    wid = s * nc + c
    base_w = wid * per_w

    def body(i, carry):
      base = base_w + i * CHUNK
      pltpu.sync_copy(ids_hbm.at[pl.ds(base, CHUNK)], idx_v)
      pltpu.async_copy(table_hbm.at[idx_v], rows_v, sem).wait()
      pltpu.sync_copy(rows_v, out_hbm.at[pl.ds(base, CHUNK)])
      return carry

    lax.fori_loop(0, n_chunks, body, 0)

  return gather_kernel


def kernel(token_ids, table):
  b, t = token_ids.shape
  flat = token_ids.reshape(b * t).astype(jnp.int32)
  out = _make_kernel(b * t)(flat, table)
  return out.reshape(b, t, DIM)


# speedup vs baseline: 1.4493x; 1.0035x over previous
import functools

import jax
import jax.numpy as jnp
from jax import lax
from jax.experimental import pallas as pl
from jax.experimental.pallas import tpu as pltpu
from jax.experimental.pallas import tpu_sc as plsc

VOCAB = 256
DIM = 768
CHUNK = 64


def _make_kernel(num_tokens: int):
  info = plsc.get_sparse_core_info()
  nc, ns = info.num_cores, info.num_subcores
  nw = nc * ns
  assert num_tokens % (nw * 2 * CHUNK) == 0
  per_w = num_tokens // nw
  n_pairs = per_w // (2 * CHUNK)

  mesh = plsc.VectorSubcoreMesh(core_axis_name="c", subcore_axis_name="s")

  @functools.partial(
      pl.kernel,
      out_type=jax.ShapeDtypeStruct((num_tokens, DIM), jnp.float32),
      mesh=mesh,
      scratch_types=[
          pltpu.VMEM((CHUNK,), jnp.int32),
          pltpu.VMEM((CHUNK,), jnp.int32),
          pltpu.VMEM((CHUNK, DIM), jnp.float32),
          pltpu.VMEM((CHUNK, DIM), jnp.float32),
          pltpu.SemaphoreType.DMA,
          pltpu.SemaphoreType.DMA,
      ],
  )
  def gather_kernel(ids_hbm, table_hbm, out_hbm,
                    idx0, idx1, rows0, rows1, sem0, sem1):
    c = lax.axis_index("c")
    s = lax.axis_index("s")
    wid = s * nc + c
    base_w = wid * per_w

    pltpu.sync_copy(ids_hbm.at[pl.ds(base_w, CHUNK)], idx0)
    g0 = pltpu.async_copy(table_hbm.at[idx0], rows0, sem0)

    def body(i, carry):
      base = base_w + i * 2 * CHUNK
      pltpu.sync_copy(ids_hbm.at[pl.ds(base + CHUNK, CHUNK)], idx1)
      pltpu.async_copy(table_hbm.at[idx1], rows1, sem1)
      pltpu.make_async_copy(table_hbm.at[idx0], rows0, sem0).wait()
      pltpu.sync_copy(rows0, out_hbm.at[pl.ds(base, CHUNK)])

      @pl.when(i < n_pairs - 1)
      def _():
        pltpu.sync_copy(ids_hbm.at[pl.ds(base + 2 * CHUNK, CHUNK)], idx0)
        pltpu.async_copy(table_hbm.at[idx0], rows0, sem0)

      pltpu.make_async_copy(table_hbm.at[idx1], rows1, sem1).wait()
      pltpu.sync_copy(rows1, out_hbm.at[pl.ds(base + CHUNK, CHUNK)])
      return carry

    lax.fori_loop(0, n_pairs, body, 0)

  return gather_kernel


def kernel(token_ids, table):
  b, t = token_ids.shape
  flat = token_ids.reshape(b * t).astype(jnp.int32)
  out = _make_kernel(b * t)(flat, table)
  return out.reshape(b, t, DIM)


# speedup vs baseline: 1.4504x; 1.0008x over previous
import functools

import jax
import jax.numpy as jnp
from jax import lax
from jax.experimental import pallas as pl
from jax.experimental.pallas import tpu as pltpu
from jax.experimental.pallas import tpu_sc as plsc

VOCAB = 256
DIM = 768
CHUNK = 64


def _make_kernel(num_tokens: int):
  info = plsc.get_sparse_core_info()
  nc, ns = info.num_cores, info.num_subcores
  nw = nc * ns
  assert num_tokens % (nw * 2 * CHUNK) == 0
  per_w = num_tokens // nw
  n_pairs = per_w // (2 * CHUNK)

  mesh = plsc.VectorSubcoreMesh(core_axis_name="c", subcore_axis_name="s")

  @functools.partial(
      pl.kernel,
      out_type=jax.ShapeDtypeStruct((num_tokens, DIM), jnp.float32),
      mesh=mesh,
      scratch_types=[
          pltpu.VMEM((per_w,), jnp.int32),
          pltpu.VMEM((CHUNK, DIM), jnp.float32),
          pltpu.VMEM((CHUNK, DIM), jnp.float32),
          pltpu.SemaphoreType.DMA,
          pltpu.SemaphoreType.DMA,
      ],
      compiler_params=pltpu.CompilerParams(use_tc_tiling_on_sc=True),
  )
  def gather_kernel(ids_hbm, table_hbm, out_hbm,
                    idx_blk, rows0, rows1, sem0, sem1):
    c = lax.axis_index("c")
    s = lax.axis_index("s")
    wid = s * nc + c
    base_w = wid * per_w

    pltpu.sync_copy(ids_hbm.at[pl.ds(base_w, per_w)], idx_blk)

    pltpu.async_copy(table_hbm.at[idx_blk.at[pl.ds(0, CHUNK)]], rows0, sem0)

    def body(i, carry):
      o = i * 2 * CHUNK
      pltpu.async_copy(
          table_hbm.at[idx_blk.at[pl.ds(o + CHUNK, CHUNK)]], rows1, sem1)
      pltpu.make_async_copy(
          table_hbm.at[idx_blk.at[pl.ds(o, CHUNK)]], rows0, sem0).wait()
      pltpu.sync_copy(rows0, out_hbm.at[pl.ds(base_w + o, CHUNK)])

      @pl.when(i < n_pairs - 1)
      def _():
        pltpu.async_copy(
            table_hbm.at[idx_blk.at[pl.ds(o + 2 * CHUNK, CHUNK)]], rows0, sem0)

      pltpu.make_async_copy(
          table_hbm.at[idx_blk.at[pl.ds(o + CHUNK, CHUNK)]], rows1, sem1).wait()
      pltpu.sync_copy(rows1, out_hbm.at[pl.ds(base_w + o + CHUNK, CHUNK)])
      return carry

    lax.fori_loop(0, n_pairs, body, 0)

  return gather_kernel


def kernel(token_ids, table):
  b, t = token_ids.shape
  flat = token_ids.reshape(b * t).astype(jnp.int32)
  out = _make_kernel(b * t)(flat, table)
  return out.reshape(b, t, DIM)


# speedup vs baseline: 1.4922x; 1.0289x over previous
import functools

import jax
import jax.numpy as jnp
from jax import lax
from jax.experimental import pallas as pl
from jax.experimental.pallas import tpu as pltpu
from jax.experimental.pallas import tpu_sc as plsc

VOCAB = 256
DIM = 768
CHUNK = 64


def _make_kernel(num_tokens: int):
  info = plsc.get_sparse_core_info()
  nc, ns = info.num_cores, info.num_subcores
  nw = nc * ns
  assert num_tokens % (nw * 2 * CHUNK) == 0
  per_w = num_tokens // nw
  n_pairs = per_w // (2 * CHUNK)

  mesh = plsc.VectorSubcoreMesh(core_axis_name="c", subcore_axis_name="s")

  @functools.partial(
      pl.kernel,
      out_type=jax.ShapeDtypeStruct((num_tokens, DIM), jnp.float32),
      mesh=mesh,
      scratch_types=[
          pltpu.VMEM((per_w,), jnp.int32),
          pltpu.VMEM((CHUNK, DIM), jnp.float32),
          pltpu.VMEM((CHUNK, DIM), jnp.float32),
          pltpu.SemaphoreType.DMA,
          pltpu.SemaphoreType.DMA,
      ],
      compiler_params=pltpu.CompilerParams(use_tc_tiling_on_sc=True),
  )
  def gather_kernel(ids_hbm, table_hbm, out_hbm,
                    idx_blk, rows0, rows1, sem0, sem1):
    c = lax.axis_index("c")
    s = lax.axis_index("s")
    wid = s * nc + c
    base_w = wid * per_w

    pltpu.sync_copy(ids_hbm.at[pl.ds(base_w, per_w)], idx_blk)

    pltpu.async_copy(table_hbm.at[idx_blk.at[pl.ds(0, CHUNK)]], rows0, sem0)

    def body(i, carry):
      o = i * 2 * CHUNK
      pltpu.async_copy(
          table_hbm.at[idx_blk.at[pl.ds(o + CHUNK, CHUNK)]], rows1, sem1)
      pltpu.make_async_copy(
          table_hbm.at[idx_blk.at[pl.ds(o, CHUNK)]], rows0, sem0).wait()
      pltpu.sync_copy(rows0, out_hbm.at[pl.ds(base_w + o, CHUNK)])

      @pl.when(i < n_pairs - 1)
      def _():
        pltpu.async_copy(
            table_hbm.at[idx_blk.at[pl.ds(o + 2 * CHUNK, CHUNK)]], rows0, sem0)

      pltpu.make_async_copy(
          table_hbm.at[idx_blk.at[pl.ds(o + CHUNK, CHUNK)]], rows1, sem1).wait()
      pltpu.sync_copy(rows1, out_hbm.at[pl.ds(base_w + o + CHUNK, CHUNK)])
      return carry

    lax.fori_loop(0, n_pairs, body, 0)

  return gather_kernel


GRP = 8


def _relayout_body(x_ref, y_ref):
  seq = y_ref.shape[1]
  for j in range(GRP):
    y_ref[j] = x_ref[pl.ds(j * seq, seq), :]


def _relayout(x, batch: int, seq: int):
  return pl.pallas_call(
      _relayout_body,
      out_shape=jax.ShapeDtypeStruct((batch, seq, DIM), jnp.float32),
      in_specs=[pl.BlockSpec((GRP * seq, DIM), lambda g: (g, 0))],
      out_specs=pl.BlockSpec((GRP, seq, DIM), lambda g: (g, 0, 0)),
      grid=(batch // GRP,),
  )(x)


def kernel(token_ids, table):
  b, t = token_ids.shape
  flat = token_ids.reshape(b * t).astype(jnp.int32)
  out = _make_kernel(b * t)(flat, table)
  return _relayout(out, b, t)
